# baseline (speedup 1.0000x reference)
import functools

import jax
import jax.numpy as jnp
from jax import lax
from jax.experimental import pallas as pl
from jax.experimental.pallas import tpu as pltpu
from jax.experimental.pallas import tpu_sc as plsc

CUT = 15
HID = 768
B = 256
S = 576

_info = plsc.get_sparse_core_info()
_NC, _NS = _info.num_cores, _info.num_subcores
_NW = _NC * _NS
_BPW = B // _NW


def _sc_lookup(h_table, w_table, idx0, idx1):
    mesh = plsc.VectorSubcoreMesh(core_axis_name="c", subcore_axis_name="s")

    @functools.partial(
        pl.kernel,
        mesh=mesh,
        out_type=[
            jax.ShapeDtypeStruct((B, HID), jnp.float32),
            jax.ShapeDtypeStruct((B, HID), jnp.float32),
        ],
        scratch_types=[
            pltpu.VMEM((_BPW,), jnp.int32),
            pltpu.VMEM((_BPW,), jnp.int32),
            pltpu.VMEM((_BPW, HID), jnp.float32),
            pltpu.VMEM((_BPW, HID), jnp.float32),
            pltpu.SemaphoreType.DMA,
            pltpu.SemaphoreType.DMA,
        ],
    )
    def k(h_hbm, w_hbm, i0_hbm, i1_hbm, oh_hbm, ow_hbm,
          i0_v, i1_v, hr_v, wr_v, s0, s1):
        wid = lax.axis_index("s") * _NC + lax.axis_index("c")
        base = wid * _BPW
        pltpu.sync_copy(i0_hbm.at[pl.ds(base, _BPW)], i0_v)
        pltpu.sync_copy(i1_hbm.at[pl.ds(base, _BPW)], i1_v)
        c0 = pltpu.async_copy(h_hbm.at[i0_v], hr_v, s0)
        c1 = pltpu.async_copy(w_hbm.at[i1_v], wr_v, s1)
        c0.wait()
        c1.wait()
        pltpu.sync_copy(hr_v, oh_hbm.at[pl.ds(base, _BPW)])
        pltpu.sync_copy(wr_v, ow_hbm.at[pl.ds(base, _BPW)])

    return k(h_table, w_table, idx0, idx1)


_BB = 8


def _tc_body(h_ref, hr_ref, wr_ref, o_ref):
    pe = hr_ref[0] + wr_ref[0]
    o_ref[...] = h_ref[...] + pe


def _tc_add(hidden, h_rows, w_rows):
    hr = h_rows.reshape(B // _BB, _BB, 1, HID)
    wr = w_rows.reshape(B // _BB, _BB, 1, HID)
    return pl.pallas_call(
        _tc_body,
        grid=(B // _BB,),
        in_specs=[
            pl.BlockSpec((_BB, S, HID), lambda b: (b, 0, 0)),
            pl.BlockSpec((1, _BB, 1, HID), lambda b: (b, 0, 0, 0)),
            pl.BlockSpec((1, _BB, 1, HID), lambda b: (b, 0, 0, 0)),
        ],
        out_specs=pl.BlockSpec((_BB, S, HID), lambda b: (b, 0, 0)),
        out_shape=jax.ShapeDtypeStruct((B, S, HID), jnp.float32),
    )(hidden, hr, wr)


def _tc_add_slice(hidden, hr, wr):
    nb = hidden.shape[0]
    return pl.pallas_call(
        _tc_body,
        grid=(nb // _BB,),
        in_specs=[
            pl.BlockSpec((_BB, S, HID), lambda b: (b, 0, 0)),
            pl.BlockSpec((1, _BB, 1, HID), lambda b: (b, 0, 0, 0)),
            pl.BlockSpec((1, _BB, 1, HID), lambda b: (b, 0, 0, 0)),
        ],
        out_specs=pl.BlockSpec((_BB, S, HID), lambda b: (b, 0, 0)),
        out_shape=jax.ShapeDtypeStruct((nb, S, HID), jnp.float32),
    )(hidden, hr, wr)


def kernel(hidden_states, patch_positions, h_table, w_table):
    idx0 = patch_positions[:, 0].astype(jnp.int32)
    idx1 = patch_positions[:, 1].astype(jnp.int32)
    h_rows, w_rows = _sc_lookup(h_table, w_table, idx0, idx1)
    hr = h_rows.reshape(B // _BB, _BB, 1, HID)
    wr = w_rows.reshape(B // _BB, _BB, 1, HID)
    half = B // 2
    hb = half // _BB
    out_a = _tc_add_slice(hidden_states[:half], hr[:hb], wr[:hb])
    out_b = _tc_add_slice(hidden_states[half:], hr[hb:], wr[hb:])
    return jnp.concatenate([out_a, out_b], axis=0)


# speedup vs baseline: 2.7981x; 2.7981x over previous
import functools

import jax
import jax.numpy as jnp
from jax import lax
from jax.experimental import pallas as pl
from jax.experimental.pallas import tpu as pltpu
from jax.experimental.pallas import tpu_sc as plsc

CUT = 15
HID = 768
B = 256
S = 576

_info = plsc.get_sparse_core_info()
_NC, _NS = _info.num_cores, _info.num_subcores
_NW = _NC * _NS
_BPW = B // _NW


def _sc_lookup(h_table, w_table, idx0, idx1):
    mesh = plsc.VectorSubcoreMesh(core_axis_name="c", subcore_axis_name="s")

    @functools.partial(
        pl.kernel,
        mesh=mesh,
        out_type=[
            jax.ShapeDtypeStruct((B, HID), jnp.float32),
            jax.ShapeDtypeStruct((B, HID), jnp.float32),
        ],
        scratch_types=[
            pltpu.VMEM((_BPW,), jnp.int32),
            pltpu.VMEM((_BPW,), jnp.int32),
            pltpu.VMEM((_BPW, HID), jnp.float32),
            pltpu.VMEM((_BPW, HID), jnp.float32),
            pltpu.SemaphoreType.DMA,
            pltpu.SemaphoreType.DMA,
        ],
    )
    def k(h_hbm, w_hbm, i0_hbm, i1_hbm, oh_hbm, ow_hbm,
          i0_v, i1_v, hr_v, wr_v, s0, s1):
        wid = lax.axis_index("s") * _NC + lax.axis_index("c")
        base = wid * _BPW
        pltpu.sync_copy(i0_hbm.at[pl.ds(base, _BPW)], i0_v)
        pltpu.sync_copy(i1_hbm.at[pl.ds(base, _BPW)], i1_v)
        c0 = pltpu.async_copy(h_hbm.at[i0_v], hr_v, s0)
        c1 = pltpu.async_copy(w_hbm.at[i1_v], wr_v, s1)
        c0.wait()
        c1.wait()
        pltpu.sync_copy(hr_v, oh_hbm.at[pl.ds(base, _BPW)])
        pltpu.sync_copy(wr_v, ow_hbm.at[pl.ds(base, _BPW)])

    return k(h_table, w_table, idx0, idx1)


_BB = 8
_SCK = 192


def _tc_body(h_ref, hr_ref, wr_ref, o_ref):
    pe = hr_ref[0] + wr_ref[0]
    o_ref[...] = h_ref[...] + pe


def _tc_add(hidden, h_rows, w_rows):
    hr = h_rows.reshape(B // _BB, _BB, 1, HID)
    wr = w_rows.reshape(B // _BB, _BB, 1, HID)
    return pl.pallas_call(
        _tc_body,
        grid=(B // _BB, S // _SCK),
        in_specs=[
            pl.BlockSpec((_BB, _SCK, HID), lambda b, s: (b, s, 0)),
            pl.BlockSpec((1, _BB, 1, HID), lambda b, s: (b, 0, 0, 0)),
            pl.BlockSpec((1, _BB, 1, HID), lambda b, s: (b, 0, 0, 0)),
        ],
        out_specs=pl.BlockSpec((_BB, _SCK, HID), lambda b, s: (b, s, 0)),
        out_shape=jax.ShapeDtypeStruct((B, S, HID), jnp.float32),
    )(hidden, hr, wr)


def kernel(hidden_states, patch_positions, h_table, w_table):
    idx0 = patch_positions[:, 0].astype(jnp.int32)
    idx1 = patch_positions[:, 1].astype(jnp.int32)
    h_rows, w_rows = _sc_lookup(h_table, w_table, idx0, idx1)
    return _tc_add(hidden_states, h_rows, w_rows)


# speedup vs baseline: 2.8814x; 1.0298x over previous
import functools

import jax
import jax.numpy as jnp
from jax import lax
from jax.experimental import pallas as pl
from jax.experimental.pallas import tpu as pltpu
from jax.experimental.pallas import tpu_sc as plsc

CUT = 15
HID = 768
B = 256
S = 576

_info = plsc.get_sparse_core_info()
_NC, _NS = _info.num_cores, _info.num_subcores
_NW = _NC * _NS
_BPW = B // _NW


def _sc_lookup(h_table, w_table, idx0, idx1):
    mesh = plsc.VectorSubcoreMesh(core_axis_name="c", subcore_axis_name="s")

    @functools.partial(
        pl.kernel,
        mesh=mesh,
        out_type=[
            jax.ShapeDtypeStruct((B, HID), jnp.float32),
            jax.ShapeDtypeStruct((B, HID), jnp.float32),
        ],
        scratch_types=[
            pltpu.VMEM((_BPW,), jnp.int32),
            pltpu.VMEM((_BPW,), jnp.int32),
            pltpu.VMEM((_BPW, HID), jnp.float32),
            pltpu.VMEM((_BPW, HID), jnp.float32),
            pltpu.SemaphoreType.DMA,
            pltpu.SemaphoreType.DMA,
        ],
    )
    def k(h_hbm, w_hbm, i0_hbm, i1_hbm, oh_hbm, ow_hbm,
          i0_v, i1_v, hr_v, wr_v, s0, s1):
        wid = lax.axis_index("s") * _NC + lax.axis_index("c")
        base = wid * _BPW
        pltpu.sync_copy(i0_hbm.at[pl.ds(base, _BPW)], i0_v)
        pltpu.sync_copy(i1_hbm.at[pl.ds(base, _BPW)], i1_v)
        c0 = pltpu.async_copy(h_hbm.at[i0_v], hr_v, s0)
        c1 = pltpu.async_copy(w_hbm.at[i1_v], wr_v, s1)
        c0.wait()
        c1.wait()
        pltpu.sync_copy(hr_v, oh_hbm.at[pl.ds(base, _BPW)])
        pltpu.sync_copy(wr_v, ow_hbm.at[pl.ds(base, _BPW)])

    return k(h_table, w_table, idx0, idx1)


_BB = 8


def _tc_body(h_ref, hr_ref, wr_ref, o_ref):
    base = pl.multiple_of(pl.program_id(0) * _BB, _BB)
    pe = hr_ref[pl.ds(base, _BB), :] + wr_ref[pl.ds(base, _BB), :]
    o_ref[...] = h_ref[...] + pe[:, None, :]


def _tc_add(hidden, h_rows, w_rows):
    return pl.pallas_call(
        _tc_body,
        grid=(B // _BB,),
        in_specs=[
            pl.BlockSpec((_BB, S, HID), lambda b: (b, 0, 0)),
            pl.BlockSpec((B, HID), lambda b: (0, 0)),
            pl.BlockSpec((B, HID), lambda b: (0, 0)),
        ],
        out_specs=pl.BlockSpec((_BB, S, HID), lambda b: (b, 0, 0)),
        out_shape=jax.ShapeDtypeStruct((B, S, HID), jnp.float32),
    )(hidden, h_rows, w_rows)


def kernel(hidden_states, patch_positions, h_table, w_table):
    idx0 = patch_positions[:, 0].astype(jnp.int32)
    idx1 = patch_positions[:, 1].astype(jnp.int32)
    h_rows, w_rows = _sc_lookup(h_table, w_table, idx0, idx1)
    return _tc_add(hidden_states, h_rows, w_rows)
